# baseline (speedup 1.0000x reference)
import functools
import math

import jax
import jax.numpy as jnp
from jax import lax
from jax.experimental import pallas as pl
from jax.experimental.pallas import tpu as pltpu
from jax.experimental.pallas import tpu_sc as plsc

ENTITY_NUM = 100000
RELATION_NUM = 1000
DIM = 64
HD = DIM // 2
BATCH = 16384
PATHS_PER_PAIR = 4
PATH_LEN = 3
GAMMA = 1.0

NC = 2
NS = 16
L = 16
NW = NC * NS
W = BATCH // NW
C = 128
NCHUNK = W // C
CP = C * PATHS_PER_PAIR
CT = C * PATHS_PER_PAIR * PATH_LEN

_ROWBUFS = ("phb", "ptb", "nhb", "ntb")
_IDXBUFS = ("phv", "ptv", "nhv", "ntv")


def _fast_sqrt(s):
    x = jnp.maximum(s, 1e-30)
    i = plsc.bitcast(x, jnp.int32)
    i = jnp.full((L,), 0x5F3759DF, jnp.int32) - lax.shift_right_logical(i, 1)
    y = plsc.bitcast(i, jnp.float32)
    half = 0.5 * x
    for _ in range(3):
        y = y * (1.5 - half * y * y)
    return x * y


def _unpack(v):
    lo = plsc.bitcast(lax.shift_left(v, 16), jnp.float32)
    hi = plsc.bitcast(
        jnp.bitwise_and(v, jnp.full((L,), -65536, jnp.int32)), jnp.float32)
    return lo, hi


def _body(et_hbm, rel_hbm, probs_hbm,
          ph_hbm, pr_hbm, pt_hbm, nh_hbm, nr_hbm, nt_hbm, tok_hbm,
          out_hbm, relv, accv, *sbufs):
    cid = lax.axis_index("c")
    sid = lax.axis_index("s")
    wid = sid * NC + cid

    names = _ROWBUFS + _IDXBUFS + ("prv", "nrv", "tokv", "probv", "sem")
    n = len(names)
    sets = [dict(zip(names, sbufs[:n])), dict(zip(names, sbufs[n:]))]

    accv[...] = jnp.zeros((L,), jnp.float32)
    pltpu.sync_copy(rel_hbm, relv)

    def dma_pairs(base, b):
        return [
            (pr_hbm.at[pl.ds(base, C)], b["prv"]),
            (nr_hbm.at[pl.ds(base, C)], b["nrv"]),
            (tok_hbm.at[pl.ds(base * 12, CT)], b["tokv"]),
            (probs_hbm.at[pl.ds(base * 4, CP)], b["probv"]),
            (et_hbm.at[b["phv"]], b["phb"]),
            (et_hbm.at[b["ptv"]], b["ptb"]),
            (et_hbm.at[b["nhv"]], b["nhb"]),
            (et_hbm.at[b["ntv"]], b["ntb"]),
        ]

    def issue(j, b):
        base = wid * W + j * C
        pltpu.sync_copy(ph_hbm.at[pl.ds(base, C)], b["phv"])
        pltpu.sync_copy(pt_hbm.at[pl.ds(base, C)], b["ptv"])
        pltpu.sync_copy(nh_hbm.at[pl.ds(base, C)], b["nhv"])
        pltpu.sync_copy(nt_hbm.at[pl.ds(base, C)], b["ntv"])
        for s, d in dma_pairs(base, b):
            pltpu.async_copy(s, d, b["sem"])

    def drain(j, b):
        base = wid * W + j * C
        for s, d in dma_pairs(base, b):
            pltpu.make_async_copy(s, d, b["sem"]).wait()

    def compute(b):
        def group_body(g, loss16):
            lane = lax.iota(jnp.int32, 16)
            rl = lane + g * L
            rl4 = rl * 4
            rl12 = rl * 12
            pr0 = plsc.load_gather(b["probv"], [rl4])
            pr1 = plsc.load_gather(b["probv"], [rl4 + 1])
            pr2 = plsc.load_gather(b["probv"], [rl4 + 2])
            pr3 = plsc.load_gather(b["probv"], [rl4 + 3])
            pridx = plsc.load_gather(b["prv"], [rl])
            nridx = plsc.load_gather(b["nrv"], [rl])
            trow = [plsc.load_gather(b["tokv"], [rl12 + k]) for k in range(12)]

            z = jnp.zeros((L,), jnp.float32)

            @plsc.parallel_loop(0, HD, 1, unroll=2, carry=(z, z))
            def c_loop(c, carry):
                s_pos, s_neg = carry
                cv = jnp.bitwise_and(c + lane, HD - 1)
                cvt = cv + HD
                eh = _unpack(plsc.load_gather(b["phb"], [rl, cv]))
                th = _unpack(plsc.load_gather(b["phb"], [rl, cvt]))
                et = _unpack(plsc.load_gather(b["ptb"], [rl, cv]))
                tt = _unpack(plsc.load_gather(b["ptb"], [rl, cvt]))
                neh = _unpack(plsc.load_gather(b["nhb"], [rl, cv]))
                nth = _unpack(plsc.load_gather(b["nhb"], [rl, cvt]))
                net = _unpack(plsc.load_gather(b["ntb"], [rl, cv]))
                ntt = _unpack(plsc.load_gather(b["ntb"], [rl, cvt]))
                rp = _unpack(plsc.load_gather(relv, [pridx, cv]))
                nr = _unpack(plsc.load_gather(relv, [nridx, cv]))
                t = [_unpack(plsc.load_gather(relv, [trow[k], cv]))
                     for k in range(12)]
                for h in range(2):
                    s0 = t[0][h] + t[1][h] + t[2][h]
                    s1 = t[3][h] + t[4][h] + t[5][h]
                    s2 = t[6][h] + t[7][h] + t[8][h]
                    s3 = t[9][h] + t[10][h] + t[11][h]
                    pf = pr0 * s0 + pr1 * s1 + pr2 * s2 + pr3 * s3
                    pos = eh[h] * th[h] + rp[h] + pf - et[h] * tt[h]
                    neg = neh[h] * nth[h] + nr[h] - net[h] * ntt[h]
                    s_pos = s_pos + pos * pos
                    s_neg = s_neg + neg * neg
                return s_pos, s_neg

            s_pos, s_neg = c_loop
            pn = _fast_sqrt(s_pos)
            nn = _fast_sqrt(s_neg)
            dd = GAMMA + pn - nn
            return loss16 + jnp.maximum(dd, 0.0) + 0.001 * (pn + nn)

        loss16 = lax.fori_loop(0, C // L, group_body,
                               jnp.zeros((L,), jnp.float32))
        accv[...] = accv[...] + loss16

    issue(0, sets[0])

    def body2(k, _):
        issue(2 * k + 1, sets[1])
        drain(2 * k, sets[0])
        compute(sets[0])

        @pl.when(k < NCHUNK // 2 - 1)
        def _issue_next():
            issue(2 * k + 2, sets[0])

        drain(2 * k + 1, sets[1])
        compute(sets[1])
        return 0

    lax.fori_loop(0, NCHUNK // 2, body2, 0)
    pltpu.sync_copy(accv, out_hbm.at[pl.ds(wid * L, L)])


def _pack(tab):
    b = tab.astype(jnp.bfloat16).reshape(-1, HD, 2)
    return lax.bitcast_convert_type(b, jnp.int32)


@jax.jit
def _run(entity_emb, relation_emb, type_emb, path_probs,
         pos_head, pos_relation, pos_tail,
         neg_head, neg_relation, neg_tail, path_rel_idx):
    et_packed = jnp.concatenate([_pack(entity_emb), _pack(type_emb)], axis=1)
    rel_packed = _pack(relation_emb)

    mesh = plsc.VectorSubcoreMesh(core_axis_name="c", subcore_axis_name="s",
                                  num_cores=NC, num_subcores=NS)
    one_set = (
        [pltpu.VMEM((C, DIM), jnp.int32)] * len(_ROWBUFS)
        + [pltpu.VMEM((C,), jnp.int32)] * len(_IDXBUFS)
        + [pltpu.VMEM((C,), jnp.int32),
           pltpu.VMEM((C,), jnp.int32),
           pltpu.VMEM((CT,), jnp.int32),
           pltpu.VMEM((CP,), jnp.float32),
           pltpu.SemaphoreType.DMA]
    )
    kern = pl.kernel(
        _body,
        out_type=jax.ShapeDtypeStruct((NW * L,), jnp.float32),
        mesh=mesh,
        compiler_params=pltpu.CompilerParams(
            needs_layout_passes=False, use_tc_tiling_on_sc=False),
        scratch_types=(
            [pltpu.VMEM((RELATION_NUM, HD), jnp.int32),
             pltpu.VMEM((L,), jnp.float32)]
            + one_set + one_set
        ),
    )
    partials = kern(et_packed, rel_packed, path_probs,
                    pos_head, pos_relation, pos_tail,
                    neg_head, neg_relation, neg_tail, path_rel_idx)
    return jnp.sum(partials)


def kernel(entity_emb, relation_emb, type_emb, path_probs,
           pos_head, pos_relation, pos_tail,
           neg_head, neg_relation, neg_tail, path_rel_idx):
    return _run(entity_emb, relation_emb, type_emb, path_probs,
                pos_head.astype(jnp.int32), pos_relation.astype(jnp.int32),
                pos_tail.astype(jnp.int32), neg_head.astype(jnp.int32),
                neg_relation.astype(jnp.int32), neg_tail.astype(jnp.int32),
                path_rel_idx.astype(jnp.int32))
